# baseline (speedup 1.0000x reference)
import functools

import jax
import jax.numpy as jnp
from jax import lax
from jax.experimental import pallas as pl
from jax.experimental.pallas import tpu as pltpu
from jax.experimental.pallas import tpu_sc as plsc

_NBFEAT = 128
_L = 16
_NC = 2
_NS = 16
_NW = _NC * _NS
_PREFIX = 808
_R = 64
_NGROUPS = _NBFEAT // _L


def _sc_qtransform(x2, cl, ch, w, n_rows):
    rows_per_w = n_rows // _NW
    n_chunks = rows_per_w // _R
    mesh = plsc.VectorSubcoreMesh(
        core_axis_name="c", subcore_axis_name="s",
        num_cores=_NC, num_subcores=_NS)

    def body(x_hbm, cl_hbm, ch_hbm, w_hbm, out_hbm,
             cl_v, ch_v, w_v, buf0, buf1, ob0, ob1,
             isem0, isem1, osem0, osem1):
        wid = lax.axis_index("c") * _NS + lax.axis_index("s")
        base = wid * rows_per_w

        pltpu.sync_copy(cl_hbm, cl_v)
        pltpu.sync_copy(ch_hbm, ch_v)
        pltpu.sync_copy(w_hbm, w_v)

        clv = [cl_v[pl.ds(g * _L, _L)] for g in range(_NGROUPS)]
        chv = [ch_v[pl.ds(g * _L, _L)] for g in range(_NGROUPS)]
        wv = [w_v[pl.ds(g * _L, _L)] for g in range(_NGROUPS)]

        bufs = [buf0, buf1]
        obs = [ob0, ob1]
        isems = [isem0, isem1]
        osems = [osem0, osem1]

        def in_copy(c):
            return pltpu.make_async_copy(
                x_hbm.at[pl.ds(base + c * _R, _R), pl.ds(0, _PREFIX)],
                bufs[c % 2], isems[c % 2])

        def out_copy(c):
            return pltpu.make_async_copy(
                obs[c % 2], out_hbm.at[pl.ds(base + c * _R, _R)],
                osems[c % 2])

        in_copy(0).start()
        for c in range(n_chunks):
            cur = c % 2
            if c + 1 < n_chunks:
                in_copy(c + 1).start()
            in_copy(c).wait()
            if c >= 2:
                out_copy(c - 2).wait()
            buf, ob = bufs[cur], obs[cur]

            def row_body(r, _):
                rb = jnp.full((_L,), r, dtype=jnp.int32)
                for g in range(_NGROUPS):
                    lo = plsc.load_gather(buf, [rb, clv[g]])
                    hi = plsc.load_gather(buf, [rb, chv[g]])
                    ob[r, pl.ds(g * _L, _L)] = lo + wv[g] * (hi - lo)
                return 0

            lax.fori_loop(0, _R, row_body, 0)
            out_copy(c).start()
        out_copy(n_chunks - 2).wait()
        out_copy(n_chunks - 1).wait()

    call = pl.kernel(
        body,
        out_type=jax.ShapeDtypeStruct((n_rows, _NBFEAT), jnp.float32),
        mesh=mesh,
        compiler_params=pltpu.CompilerParams(
            needs_layout_passes=False, use_tc_tiling_on_sc=False),
        scratch_types=[
            pltpu.VMEM((_NBFEAT,), jnp.int32),
            pltpu.VMEM((_NBFEAT,), jnp.int32),
            pltpu.VMEM((_NBFEAT,), jnp.float32),
            pltpu.VMEM((_R, _PREFIX), jnp.float32),
            pltpu.VMEM((_R, _PREFIX), jnp.float32),
            pltpu.VMEM((_R, _NBFEAT), jnp.float32),
            pltpu.VMEM((_R, _NBFEAT), jnp.float32),
            pltpu.SemaphoreType.DMA,
            pltpu.SemaphoreType.DMA,
            pltpu.SemaphoreType.DMA,
            pltpu.SemaphoreType.DMA,
        ],
    )
    return call(x2, cl, ch, w)


def kernel(input):
    x = input
    b, t, c = x.shape
    n_rows = b * t
    halftone = jnp.float32(2.0 ** (1.0 / 12.0))
    f0 = jnp.float32(440.0 / 16000.0 * 1024.0)
    freq = f0 * jnp.power(halftone, jnp.arange(_NBFEAT, dtype=jnp.float32) - 69.0)
    lowfreq = jnp.floor(freq)
    w = freq - lowfreq
    cl = lowfreq.astype(jnp.int32)
    ch = cl + 1
    x2 = x.reshape(n_rows, c)
    out2 = _sc_qtransform(x2, cl, ch, w, n_rows)
    return out2.reshape(b, t, _NBFEAT)


# speedup vs baseline: 2.4157x; 2.4157x over previous
import functools

import jax
import jax.numpy as jnp
from jax import lax
from jax.experimental import pallas as pl
from jax.experimental.pallas import tpu as pltpu
from jax.experimental.pallas import tpu_sc as plsc

_NBFEAT = 128
_L = 16
_NC = 2
_NS = 16
_NW = _NC * _NS
_PREFIX = 896
_R = 32
_NGROUPS = _NBFEAT // _L


def _sc_qtransform(x2, cl, ch, w, n_rows):
    rows_per_w = n_rows // _NW
    n_chunks = rows_per_w // _R
    mesh = plsc.VectorSubcoreMesh(
        core_axis_name="c", subcore_axis_name="s",
        num_cores=_NC, num_subcores=_NS)

    def body(x_hbm, cl_hbm, ch_hbm, w_hbm, out_hbm,
             cl_v, ch_v, w_v, buf0, buf1, ob0, ob1,
             isem0, isem1, osem0, osem1):
        wid = lax.axis_index("c") * _NS + lax.axis_index("s")
        base = wid * rows_per_w

        pltpu.sync_copy(cl_hbm, cl_v)
        pltpu.sync_copy(ch_hbm, ch_v)
        pltpu.sync_copy(w_hbm, w_v)

        clv = [cl_v[pl.ds(g * _L, _L)] for g in range(_NGROUPS)]
        chv = [ch_v[pl.ds(g * _L, _L)] for g in range(_NGROUPS)]
        wv = [w_v[pl.ds(g * _L, _L)] for g in range(_NGROUPS)]

        bufs = [buf0, buf1]
        obs = [ob0, ob1]
        isems = [isem0, isem1]
        osems = [osem0, osem1]

        def in_copy(c):
            return pltpu.make_async_copy(
                x_hbm.at[pl.ds(base + c * _R, _R), pl.ds(0, _PREFIX)],
                bufs[c % 2], isems[c % 2])

        def out_copy(c):
            return pltpu.make_async_copy(
                obs[c % 2], out_hbm.at[pl.ds(base + c * _R, _R)],
                osems[c % 2])

        in_copy(0).start()
        for c in range(n_chunks):
            cur = c % 2
            if c + 1 < n_chunks:
                in_copy(c + 1).start()
            in_copy(c).wait()
            if c >= 2:
                out_copy(c - 2).wait()
            buf, ob = bufs[cur], obs[cur]

            def row_body(r, _):
                rb = jnp.full((_L,), r, dtype=jnp.int32)
                los = [plsc.load_gather(buf, [rb, clv[g]])
                       for g in range(_NGROUPS)]
                his = [plsc.load_gather(buf, [rb, chv[g]])
                       for g in range(_NGROUPS)]
                for g in range(_NGROUPS):
                    ob[r, pl.ds(g * _L, _L)] = (
                        los[g] + wv[g] * (his[g] - los[g]))
                return 0

            lax.fori_loop(0, _R, row_body, 0)
            out_copy(c).start()
        out_copy(n_chunks - 2).wait()
        out_copy(n_chunks - 1).wait()

    call = pl.kernel(
        body,
        out_type=jax.ShapeDtypeStruct((n_rows, _NBFEAT), jnp.float32),
        mesh=mesh,
        compiler_params=pltpu.CompilerParams(needs_layout_passes=False),
        scratch_types=[
            pltpu.VMEM((_NBFEAT,), jnp.int32),
            pltpu.VMEM((_NBFEAT,), jnp.int32),
            pltpu.VMEM((_NBFEAT,), jnp.float32),
            pltpu.VMEM((_R, _PREFIX), jnp.float32),
            pltpu.VMEM((_R, _PREFIX), jnp.float32),
            pltpu.VMEM((_R, _NBFEAT), jnp.float32),
            pltpu.VMEM((_R, _NBFEAT), jnp.float32),
            pltpu.SemaphoreType.DMA,
            pltpu.SemaphoreType.DMA,
            pltpu.SemaphoreType.DMA,
            pltpu.SemaphoreType.DMA,
        ],
    )
    return call(x2, cl, ch, w)


def kernel(input):
    x = input
    b, t, c = x.shape
    n_rows = b * t
    halftone = jnp.float32(2.0 ** (1.0 / 12.0))
    f0 = jnp.float32(440.0 / 16000.0 * 1024.0)
    freq = f0 * jnp.power(halftone, jnp.arange(_NBFEAT, dtype=jnp.float32) - 69.0)
    lowfreq = jnp.floor(freq)
    w = freq - lowfreq
    cl = lowfreq.astype(jnp.int32)
    ch = cl + 1
    x2 = x.reshape(n_rows, c)
    out2 = _sc_qtransform(x2, cl, ch, w, n_rows)
    return out2.reshape(b, t, _NBFEAT)


# speedup vs baseline: 2.5992x; 1.0760x over previous
import functools

import numpy as np
import jax
import jax.numpy as jnp
from jax import lax
from jax.experimental import pallas as pl
from jax.experimental.pallas import tpu as pltpu
from jax.experimental.pallas import tpu_sc as plsc

_NBFEAT = 128
_L = 16
_NC = 2
_NS = 16
_NW = _NC * _NS
_PREFIX = 896
_R = 32
_NGROUPS = _NBFEAT // _L
_NBUF = 3


def _ladder():
    k = np.arange(_NBFEAT, dtype=np.float64)
    f0 = np.float64(np.float32(440.0 / 16000.0 * 1024.0))
    freq = f0 * (2.0 ** ((k - 69.0) / 12.0))
    low = np.floor(freq)
    w = (freq - low).astype(np.float32)
    cl = low.astype(np.int32)
    return cl, cl + 1, w


def _sc_qtransform(x2, n_rows):
    rows_per_w = n_rows // _NW
    n_chunks = rows_per_w // _R
    cl_host, ch_host, w_host = _ladder()
    cl = jnp.asarray(cl_host)
    ch = jnp.asarray(ch_host)
    w = jnp.asarray(w_host)
    mesh = plsc.VectorSubcoreMesh(
        core_axis_name="c", subcore_axis_name="s",
        num_cores=_NC, num_subcores=_NS)

    def body(x_hbm, cl_hbm, ch_hbm, w_hbm, out_hbm,
             cl_v, ch_v, w_v, buf0, buf1, buf2, ob0, ob1,
             isem0, isem1, isem2, osem0, osem1):
        wid = lax.axis_index("c") * _NS + lax.axis_index("s")
        base = wid * rows_per_w

        pltpu.sync_copy(cl_hbm, cl_v)
        pltpu.sync_copy(ch_hbm, ch_v)
        pltpu.sync_copy(w_hbm, w_v)

        clv = [cl_v[pl.ds(g * _L, _L)] for g in range(_NGROUPS)]
        chv = [ch_v[pl.ds(g * _L, _L)] for g in range(_NGROUPS)]
        wv = [w_v[pl.ds(g * _L, _L)] for g in range(_NGROUPS)]

        bufs = [buf0, buf1, buf2]
        obs = [ob0, ob1]
        isems = [isem0, isem1, isem2]
        osems = [osem0, osem1]

        def in_copy(c):
            return pltpu.make_async_copy(
                x_hbm.at[pl.ds(base + c * _R, _R), pl.ds(0, _PREFIX)],
                bufs[c % _NBUF], isems[c % _NBUF])

        def out_copy(c):
            return pltpu.make_async_copy(
                obs[c % 2], out_hbm.at[pl.ds(base + c * _R, _R)],
                osems[c % 2])

        in_copy(0).start()
        in_copy(1).start()
        for c in range(n_chunks):
            if c + 2 < n_chunks:
                in_copy(c + 2).start()
            in_copy(c).wait()
            if c >= 2:
                out_copy(c - 2).wait()
            buf, ob = bufs[c % _NBUF], obs[c % 2]

            def row_body(r, _):
                rb = jnp.full((_L,), r, dtype=jnp.int32)
                los = [plsc.load_gather(buf, [rb, clv[g]])
                       for g in range(_NGROUPS)]
                his = [plsc.load_gather(buf, [rb, chv[g]])
                       for g in range(_NGROUPS)]
                for g in range(_NGROUPS):
                    ob[r, pl.ds(g * _L, _L)] = (
                        los[g] + wv[g] * (his[g] - los[g]))
                return 0

            lax.fori_loop(0, _R, row_body, 0)
            out_copy(c).start()
        out_copy(n_chunks - 2).wait()
        out_copy(n_chunks - 1).wait()

    call = pl.kernel(
        body,
        out_type=jax.ShapeDtypeStruct((n_rows, _NBFEAT), jnp.float32),
        mesh=mesh,
        compiler_params=pltpu.CompilerParams(needs_layout_passes=False),
        scratch_types=[
            pltpu.VMEM((_NBFEAT,), jnp.int32),
            pltpu.VMEM((_NBFEAT,), jnp.int32),
            pltpu.VMEM((_NBFEAT,), jnp.float32),
            pltpu.VMEM((_R, _PREFIX), jnp.float32),
            pltpu.VMEM((_R, _PREFIX), jnp.float32),
            pltpu.VMEM((_R, _PREFIX), jnp.float32),
            pltpu.VMEM((_R, _NBFEAT), jnp.float32),
            pltpu.VMEM((_R, _NBFEAT), jnp.float32),
            pltpu.SemaphoreType.DMA,
            pltpu.SemaphoreType.DMA,
            pltpu.SemaphoreType.DMA,
            pltpu.SemaphoreType.DMA,
            pltpu.SemaphoreType.DMA,
        ],
    )
    return call(x2, cl, ch, w)


def kernel(input):
    x = input
    b, t, c = x.shape
    n_rows = b * t
    x2 = x.reshape(n_rows, c)
    out2 = _sc_qtransform(x2, n_rows)
    return out2.reshape(b, t, _NBFEAT)
